# baseline (speedup 1.0000x reference)
import functools

import jax
import jax.numpy as jnp
from jax import lax
from jax.experimental import pallas as pl
from jax.experimental.pallas import tpu as pltpu
from jax.experimental.pallas import tpu_sc as plsc

_ROWS = 16384
_COLS = 128
_OUT_COLS = 4
_NC = 2
_NS = 16
_NW = _NC * _NS
_RPW = _ROWS // _NW

_mesh = plsc.VectorSubcoreMesh(core_axis_name="c", subcore_axis_name="s")


@functools.partial(
    pl.kernel,
    mesh=_mesh,
    out_type=jax.ShapeDtypeStruct((_ROWS, _OUT_COLS), jnp.float32),
    scratch_types=[
        pltpu.VMEM_SHARED((_NS, _RPW, 8), jnp.float32),
        pltpu.VMEM((_RPW, _OUT_COLS), jnp.float32),
    ],
    compiler_params=pltpu.CompilerParams(
        use_tc_tiling_on_sc=False, needs_layout_passes=False
    ),
)
def _sc_gather_cols(x_hbm, out_hbm, in_s, out_v):
    cid = lax.axis_index("c")
    sid = lax.axis_index("s")
    base = (sid * _NC + cid) * _RPW

    pltpu.sync_copy(x_hbm.at[pl.ds(base, _RPW), pl.ds(0, 8)], in_s.at[sid])

    pltpu.sync_copy(in_s.at[sid, :, pl.ds(0, 2)], out_v.at[:, pl.ds(0, 2)])
    pltpu.sync_copy(in_s.at[sid, :, pl.ds(4, 1)], out_v.at[:, pl.ds(2, 1)])
    pltpu.sync_copy(in_s.at[sid, :, pl.ds(4, 1)], out_v.at[:, pl.ds(3, 1)])

    pltpu.sync_copy(out_v, out_hbm.at[pl.ds(base, _RPW), :])


def kernel(x):
    return _sc_gather_cols(x)


# speedup vs baseline: 1.2522x; 1.2522x over previous
import functools

import jax
import jax.numpy as jnp
from jax import lax
from jax.experimental import pallas as pl
from jax.experimental.pallas import tpu as pltpu
from jax.experimental.pallas import tpu_sc as plsc

_ROWS = 16384
_OUT_COLS = 4
_NC = 2
_NS = 16
_NW = _NC * _NS
_RPW = _ROWS // _NW
_L = 16
_GROUPS = _RPW * _OUT_COLS // _L

_mesh = plsc.VectorSubcoreMesh(core_axis_name="c", subcore_axis_name="s")


@functools.partial(
    pl.kernel,
    mesh=_mesh,
    out_type=jax.ShapeDtypeStruct((_ROWS, _OUT_COLS), jnp.float32),
    scratch_types=[
        pltpu.VMEM((_RPW, 8), jnp.float32),
        pltpu.VMEM((_RPW, _OUT_COLS), jnp.float32),
    ],
    compiler_params=pltpu.CompilerParams(
        use_tc_tiling_on_sc=False, needs_layout_passes=False
    ),
)
def _sc_gather_cols(x_hbm, out_hbm, in_v, out_v):
    cid = lax.axis_index("c")
    sid = lax.axis_index("s")
    base = (sid * _NC + cid) * _RPW

    pltpu.sync_copy(x_hbm.at[pl.ds(base, _RPW), pl.ds(0, 8)], in_v)

    l = lax.iota(jnp.int32, _L)
    row_off = lax.shift_right_logical(l, 2)
    j = lax.bitwise_and(l, 3)
    src_col = lax.select(j >= 2, jnp.full((_L,), 4, jnp.int32), j)

    def body(t, carry):
        rows = row_off + 4 * t
        v = plsc.load_gather(in_v, [rows, src_col])
        plsc.store_scatter(out_v, [rows, j], v)
        return carry

    lax.fori_loop(0, _GROUPS, body, 0, unroll=8)

    pltpu.sync_copy(out_v, out_hbm.at[pl.ds(base, _RPW), :])


def kernel(x):
    return _sc_gather_cols(x)


# speedup vs baseline: 4.2352x; 3.3820x over previous
import jax
import jax.numpy as jnp
from jax.experimental import pallas as pl

_ROWS = 16384
_COLS = 128
_PANELS = _ROWS // 128
_PB = 32


def _gather_cols_kernel(x_ref, o_ref):
    y = x_ref[...].reshape(_PB, 128, _COLS)
    z = jnp.concatenate(
        [y[:, :, 0:1], y[:, :, 1:2], y[:, :, 4:5], y[:, :, 4:5]], axis=2
    )
    o_ref[...] = jnp.transpose(z, (0, 2, 1))


def kernel(x):
    grid = (_PANELS // _PB,)
    t = pl.pallas_call(
        _gather_cols_kernel,
        grid=grid,
        in_specs=[pl.BlockSpec((_PB * 128, _COLS), lambda i: (i, 0))],
        out_specs=pl.BlockSpec((_PB, 4, 128), lambda i: (i, 0, 0)),
        out_shape=jax.ShapeDtypeStruct((_PANELS, 4, 128), jnp.float32),
    )(x)
    return jnp.transpose(t, (0, 2, 1)).reshape(_ROWS, 4)


# speedup vs baseline: 7.8926x; 1.8636x over previous
import jax
import jax.numpy as jnp
from jax.experimental import pallas as pl
from jax.experimental.pallas import tpu as pltpu

_ROWS = 16384
_COLS = 128
_PANELS = _ROWS // 128
_PB = 16
_STEPS = _PANELS // _PB


def _gather_cols_kernel(x_hbm, o_ref, buf, sem):
    step = pl.program_id(0)

    def start(i, slot):
        pltpu.make_async_copy(
            x_hbm.at[pl.ds(i * _PB * 128, _PB * 128), :], buf.at[slot], sem.at[slot]
        ).start()

    @pl.when(step == 0)
    def _():
        start(0, 0)

    @pl.when(step + 1 < _STEPS)
    def _():
        start(step + 1, (step + 1) % 2)

    slot = step % 2
    pltpu.make_async_copy(
        x_hbm.at[pl.ds(step * _PB * 128, _PB * 128), :], buf.at[slot], sem.at[slot]
    ).wait()

    y = buf[slot].reshape(_PB, 128, _COLS)
    z = jnp.concatenate(
        [y[:, :, 0:1], y[:, :, 1:2], y[:, :, 4:5], y[:, :, 4:5]], axis=2
    )
    o_ref[...] = jnp.transpose(z, (0, 2, 1))


def kernel(x):
    x = pltpu.with_memory_space_constraint(x, pltpu.MemorySpace.HBM)
    t = pl.pallas_call(
        _gather_cols_kernel,
        grid=(_STEPS,),
        in_specs=[pl.BlockSpec(memory_space=pl.ANY)],
        out_specs=pl.BlockSpec((_PB, 4, 128), lambda i: (i, 0, 0)),
        out_shape=jax.ShapeDtypeStruct((_PANELS, 4, 128), jnp.float32),
        scratch_shapes=[
            pltpu.VMEM((2, _PB * 128, _COLS), jnp.float32),
            pltpu.SemaphoreType.DMA((2,)),
        ],
    )(x)
    return jnp.transpose(t, (0, 2, 1)).reshape(_ROWS, 4)


# speedup vs baseline: 7.9357x; 1.0055x over previous
import jax
import jax.numpy as jnp
from jax import lax
from jax.experimental import pallas as pl
from jax.experimental.pallas import tpu as pltpu

_ROWS = 16384
_COLS = 128
_PANELS = _ROWS // 128
_PB = 32
_STEPS = _PANELS // _PB
_SRC = (0, 1, 4, 4)


def _gather_cols_kernel(x_hbm, o_ref, buf, sem):
    step = pl.program_id(0)

    def start(i, slot):
        pltpu.make_async_copy(
            x_hbm.at[pl.ds(i * _PB * 128, _PB * 128), :], buf.at[slot], sem.at[slot]
        ).start()

    @pl.when(step == 0)
    def _():
        start(0, 0)

    @pl.when(step + 1 < _STEPS)
    def _():
        start(step + 1, (step + 1) % 2)

    slot = step % 2
    pltpu.make_async_copy(
        x_hbm.at[pl.ds(step * _PB * 128, _PB * 128), :], buf.at[slot], sem.at[slot]
    ).wait()

    k_idx = lax.broadcasted_iota(jnp.int32, (4, _COLS), 1)
    c_idx = lax.broadcasted_iota(jnp.int32, (4, _COLS), 0)
    src = jnp.where(c_idx >= 2, 4, c_idx)
    sel = jnp.where(k_idx == src, 1.0, 0.0)
    ot = lax.dot_general(
        sel,
        buf[slot],
        (((1,), (1,)), ((), ())),
        preferred_element_type=jnp.float32,
    )
    for p in range(_PB):
        o_ref[p] = ot[:, p * 128 : (p + 1) * 128]


def kernel(x):
    x = pltpu.with_memory_space_constraint(x, pltpu.MemorySpace.HBM)
    t = pl.pallas_call(
        _gather_cols_kernel,
        grid=(_STEPS,),
        in_specs=[pl.BlockSpec(memory_space=pl.ANY)],
        out_specs=pl.BlockSpec((_PB, 4, 128), lambda i: (i, 0, 0)),
        out_shape=jax.ShapeDtypeStruct((_PANELS, 4, 128), jnp.float32),
        scratch_shapes=[
            pltpu.VMEM((2, _PB * 128, _COLS), jnp.float32),
            pltpu.SemaphoreType.DMA((2,)),
        ],
    )(x)
    return jnp.transpose(t, (0, 2, 1)).reshape(_ROWS, 4)
